# baseline (speedup 1.0000x reference)
import functools
import math

import jax
import jax.numpy as jnp
from jax import lax
from jax.experimental import pallas as pl
from jax.experimental.pallas import tpu as pltpu
from jax.experimental.pallas import tpu_sc as plsc


def _scale_table(table, scale):
    V, D = table.shape
    blk = 4000
    def body(t_ref, o_ref):
        o_ref[...] = t_ref[...] * scale
    return pl.pallas_call(
        body,
        grid=(V // blk,),
        in_specs=[pl.BlockSpec((blk, D), lambda i: (i, 0))],
        out_specs=pl.BlockSpec((blk, D), lambda i: (i, 0)),
        out_shape=jax.ShapeDtypeStruct((V, D), jnp.float32),
    )(table)


@functools.partial(jax.jit, static_argnames=("B", "D"))
def _sc_gather(idx2d, table, B, D):
    info = plsc.get_sparse_core_info()
    NC, NS = info.num_cores, info.num_subcores
    NW = NC * NS
    b_per_w = B // NW
    C = 128
    n_chunks = b_per_w // C
    NBUF = 5
    assert n_chunks % NBUF == 0

    mesh = plsc.VectorSubcoreMesh(core_axis_name="c", subcore_axis_name="s")

    @functools.partial(
        pl.kernel,
        mesh=mesh,
        out_type=jax.ShapeDtypeStruct((B, D), jnp.float32),
        scratch_types=(
            [pltpu.VMEM((n_chunks, C), jnp.int32)]
            + [pltpu.VMEM((C, D), jnp.float32) for _ in range(NBUF)]
            + [pltpu.SemaphoreType.DMA for _ in range(2 * NBUF)]
        ),
    )
    def k(idx_hbm, table_hbm, out_hbm, idx_v, *bufs):
        rows = bufs[:NBUF]
        gsem = bufs[NBUF:2 * NBUF]
        ssem = bufs[2 * NBUF:]
        wid = lax.axis_index("s") * NC + lax.axis_index("c")
        out_base = wid * b_per_w

        pltpu.sync_copy(idx_hbm.at[pl.ds(wid * n_chunks, n_chunks)], idx_v)

        def gather_start(g, j):
            pltpu.async_copy(table_hbm.at[idx_v.at[g]], rows[j], gsem[j])

        def gather_wait(j):
            pltpu.make_async_copy(
                table_hbm.at[idx_v.at[0]], rows[j], gsem[j]).wait()

        def scatter_start(g, j):
            pltpu.async_copy(
                rows[j], out_hbm.at[pl.ds(out_base + g * C, C)], ssem[j])

        def scatter_wait(j):
            pltpu.make_async_copy(
                rows[j], out_hbm.at[pl.ds(out_base, C)], ssem[j]).wait()

        LAG = 3

        def outer(o, carry):
            gbase = o * NBUF
            for j in range(NBUF):
                g = gbase + j
                @pl.when(o > 0)
                def _(j=j):
                    scatter_wait(j)
                gather_start(g, j)
                jp = (j + NBUF - LAG) % NBUF
                if j < LAG:
                    @pl.when(o > 0)
                    def _(g=g, jp=jp):
                        gather_wait(jp)
                        scatter_start(g - LAG, jp)
                else:
                    gather_wait(jp)
                    scatter_start(g - LAG, jp)
            return carry

        lax.fori_loop(0, n_chunks // NBUF, outer, 0)

        for t in range(LAG):
            g = n_chunks - LAG + t
            gather_wait(g % NBUF)
            scatter_start(g, g % NBUF)
        for j in range(NBUF):
            scatter_wait(j)

    return k(idx2d, table)


def kernel(tokens, table):
    Bt, T = tokens.shape
    V, D = table.shape
    B = Bt * T
    C = 128
    scaled = _scale_table(table, math.sqrt(D))
    idx2d = tokens.reshape(B // C, C).astype(jnp.int32)
    out = _sc_gather(idx2d, scaled, B=B, D=D)
    return out.reshape(Bt, T, D)


# speedup vs baseline: 1.1128x; 1.1128x over previous
import functools
import math

import jax
import jax.numpy as jnp
from jax import lax
from jax.experimental import pallas as pl
from jax.experimental.pallas import tpu as pltpu
from jax.experimental.pallas import tpu_sc as plsc


@functools.partial(jax.jit, static_argnames=("B", "D"))
def _sc_embed(idx2d, table, B, D):
    info = plsc.get_sparse_core_info()
    NC, NS, L = info.num_cores, info.num_subcores, info.num_lanes
    NW = NC * NS
    b_per_w = B // NW
    C = 128
    n_chunks = b_per_w // C
    NBUF = 5
    assert n_chunks % NBUF == 0
    scale = math.sqrt(D)

    mesh = plsc.VectorSubcoreMesh(core_axis_name="c", subcore_axis_name="s")

    @functools.partial(
        pl.kernel,
        mesh=mesh,
        out_type=jax.ShapeDtypeStruct((B, D), jnp.float32),
        scratch_types=(
            [pltpu.VMEM((n_chunks, C), jnp.int32)]
            + [pltpu.VMEM((C, D), jnp.float32) for _ in range(NBUF)]
            + [pltpu.SemaphoreType.DMA for _ in range(2 * NBUF)]
        ),
    )
    def k(idx_hbm, table_hbm, out_hbm, idx_v, *bufs):
        rows = bufs[:NBUF]
        gsem = bufs[NBUF:2 * NBUF]
        ssem = bufs[2 * NBUF:]
        wid = lax.axis_index("s") * NC + lax.axis_index("c")
        out_base = wid * b_per_w

        pltpu.sync_copy(idx_hbm.at[pl.ds(wid * n_chunks, n_chunks)], idx_v)

        def gather_start(g, j):
            pltpu.async_copy(table_hbm.at[idx_v.at[g]], rows[j], gsem[j])

        def gather_wait(j):
            pltpu.make_async_copy(
                table_hbm.at[idx_v.at[0]], rows[j], gsem[j]).wait()

        def scale_rows(j):
            r = rows[j]
            def row_body(i, carry):
                for c in range(D // L):
                    sl = pl.ds(c * L, L)
                    r[i, sl] = r[i, sl] * scale
                return carry
            lax.fori_loop(0, C, row_body, 0)

        def scatter_start(g, j):
            pltpu.async_copy(
                rows[j], out_hbm.at[pl.ds(out_base + g * C, C)], ssem[j])

        def scatter_wait(j):
            pltpu.make_async_copy(
                rows[j], out_hbm.at[pl.ds(out_base, C)], ssem[j]).wait()

        LAG = 3

        def outer(o, carry):
            gbase = o * NBUF
            for j in range(NBUF):
                g = gbase + j
                @pl.when(o > 0)
                def _(j=j):
                    scatter_wait(j)
                gather_start(g, j)
                jp = (j + NBUF - LAG) % NBUF
                if j < LAG:
                    @pl.when(o > 0)
                    def _(g=g, jp=jp):
                        gather_wait(jp)
                        scale_rows(jp)
                        scatter_start(g - LAG, jp)
                else:
                    gather_wait(jp)
                    scale_rows(jp)
                    scatter_start(g - LAG, jp)
            return carry

        lax.fori_loop(0, n_chunks // NBUF, outer, 0)

        for t in range(LAG):
            g = n_chunks - LAG + t
            gather_wait(g % NBUF)
            scale_rows(g % NBUF)
            scatter_start(g, g % NBUF)
        for j in range(NBUF):
            scatter_wait(j)

    return k(idx2d, table)


def kernel(tokens, table):
    Bt, T = tokens.shape
    V, D = table.shape
    B = Bt * T
    C = 128
    idx2d = tokens.reshape(B // C, C).astype(jnp.int32)
    out = _sc_embed(idx2d, table, B=B, D=D)
    return out.reshape(Bt, T, D)
